# baseline (speedup 1.0000x reference)
import functools

import jax
import jax.numpy as jnp
from jax import lax
from jax.experimental import pallas as pl
from jax.experimental.pallas import tpu as pltpu
from jax.experimental.pallas import tpu_sc as plsc

B, A, NB, F, R = 4, 512, 32, 128, 20
CUTOFF = 5.0
BLK = 128
EB = A // BLK
CH = BLK * NB


def _silu(x):
    return x / (1.0 + jnp.exp(-x))


def _cutoff(D):
    x = D * (1.0 / CUTOFF)
    x2 = x * x
    x4 = x2 * x2
    x8 = x4 * x4
    x9 = x8 * x
    x10 = x9 * x
    x11 = x10 * x
    out = 1.0 - 55.0 * x9 + 99.0 * x10 - 45.0 * x11
    return out * (D < CUTOFF).astype(D.dtype)


def _mlp2(x, W1, b1, W2, b2):
    h = jnp.dot(x, W1, preferred_element_type=jnp.float32) + b1
    h = _silu(h)
    return jnp.dot(h, W2, preferred_element_type=jnp.float32) + b2


def _stage1_body(a_ref, qd_ref, ql_ref,
                 Wa1, ba1, Wa2, ba2,
                 Wq1, bq1, Wq2, bq2,
                 Wm1, bm1, Wm2, bm2,
                 We1, be1, We2, be2,
                 amsij_o, qdyn_o, qlat_o, egate_o):
    a = a_ref[...]
    amsij_o[...] = _mlp2(a, Wa1[...], ba1[...], Wa2[...], ba2[...])
    q = _mlp2(a, Wq1[...], bq1[...], Wq2[...], bq2[...])
    qm = _mlp2(a, Wm1[...], bm1[...], Wm2[...], bm2[...])
    qlat_o[...] = ql_ref[...] + q
    qdyn_o[...] = qd_ref[...] + q * qm
    egate_o[...] = _mlp2(a, We1[...], be1[...], We2[...], be2[...])


def _stage1(a2, qd2, ql2, p):
    n = a2.shape[0]
    f32 = jnp.float32
    outs = [
        jax.ShapeDtypeStruct((n, F), f32),
        jax.ShapeDtypeStruct((n, F), f32),
        jax.ShapeDtypeStruct((n, 1), f32),
        jax.ShapeDtypeStruct((n, F), f32),
    ]
    wa = p['a']
    wq = p['q']
    wm = p['qm']
    we = p['e']
    return pl.pallas_call(_stage1_body, out_shape=outs)(
        a2, qd2, ql2,
        wa[0], wa[1].reshape(1, F), wa[2], wa[3].reshape(1, F),
        wq[0], wq[1].reshape(1, F), wq[2], wq[3].reshape(1, 1),
        wm[0], wm[1].reshape(1, F), wm[2], wm[3].reshape(1, F),
        we[0], we[1].reshape(1, F), we[2], we[3].reshape(1, F),
    )


def _sc_gather2(table_a, table_b, idx):
    info = plsc.get_sparse_core_info()
    nw = info.num_cores * info.num_subcores
    m = idx.shape[0]
    per_w = m // nw
    chunk = 128
    nch = per_w // chunk
    f32 = jnp.float32

    @functools.partial(
        pl.kernel,
        out_type=[jax.ShapeDtypeStruct((m, F), f32),
                  jax.ShapeDtypeStruct((m, F), f32)],
        mesh=plsc.VectorSubcoreMesh(core_axis_name="c", subcore_axis_name="s"),
        scratch_types=[
            pltpu.VMEM((per_w,), jnp.int32),
            pltpu.VMEM((2, chunk, F), f32),
            pltpu.VMEM((2, chunk, F), f32),
            pltpu.SemaphoreType.DMA,
            pltpu.SemaphoreType.DMA,
            pltpu.SemaphoreType.DMA,
            pltpu.SemaphoreType.DMA,
        ],
    )
    def k(ta, tb, ix, out_a, out_b, idx_v, rows_a, rows_b, gsem, ssem,
          gsem2, ssem2):
        wid = lax.axis_index("s") * info.num_cores + lax.axis_index("c")
        base = wid * per_w
        pltpu.sync_copy(ix.at[pl.ds(base, per_w)], idx_v)
        gsems = (gsem, gsem2)
        ssems = (ssem, ssem2)
        scat = [None, None, None, None]
        for c in range(nch):
            s = c % 2
            if scat[2 * s] is not None:
                scat[2 * s].wait()
                scat[2 * s + 1].wait()
            isl = idx_v.at[pl.ds(c * chunk, chunk)]
            cp_a = pltpu.async_copy(ta.at[isl], rows_a.at[s], gsems[s])
            cp_b = pltpu.async_copy(tb.at[isl], rows_b.at[s], gsems[s])
            osl = pl.ds(base + c * chunk, chunk)
            cp_a.wait()
            scat[2 * s] = pltpu.async_copy(rows_a.at[s], out_a.at[osl],
                                           ssems[s])
            cp_b.wait()
            scat[2 * s + 1] = pltpu.async_copy(rows_b.at[s], out_b.at[osl],
                                               ssems[s])
        for cp in scat:
            cp.wait()

    return k(table_a, table_b, idx)


def _stage3_body(a_ref, qd_ref, eg_ref, ed_ref, ai_ref,
                 aj_ref, qj_ref, rbf_ref, d_ref,
                 bd_ref, bl_ref,
                 Wr, br, Wb1, bb1, Wb2, bb2, Wm1, bm1, Wm2, bm2,
                 aout_o, eout_o, bdyn_o, blat_o):
    d2 = d_ref[0]
    cut3 = _cutoff(d2)[:, :, None]
    dinv3 = jnp.where(d2 > 0.0, 1.0 / d2, 0.0)[:, :, None]
    rbfm = jnp.dot(rbf_ref[0].reshape(CH, R), Wr[...],
                   preferred_element_type=jnp.float32) + br[...]
    ai3 = ai_ref[...].reshape(BLK, 1, F)
    msij3 = ai3 * aj_ref[...].reshape(BLK, NB, F) * (rbfm.reshape(BLK, NB, F) * cut3)
    msij = msij3.reshape(CH, F)
    bf16 = jnp.bfloat16
    msij_h = msij.astype(bf16)
    h = _silu(jnp.dot(msij_h, Wb1[...].astype(bf16),
                      preferred_element_type=jnp.float32) + bb1[...])
    h2 = _silu(jnp.dot(msij_h, Wm1[...].astype(bf16),
                       preferred_element_type=jnp.float32) + bm1[...])
    bij = jnp.dot(h.astype(bf16), Wb2[...].astype(bf16),
                  preferred_element_type=jnp.float32) + bb2[...]
    bm = jnp.dot(h2.astype(bf16), Wm2[...].astype(bf16),
                 preferred_element_type=jnp.float32) + bm2[...]
    bnew3 = bd_ref[0] + (bij * bm).reshape(BLK, NB, F)
    bdyn_o[0] = bnew3
    blat_o[0] = bl_ref[0] + bij.reshape(BLK, NB)
    qi3 = qd_ref[...].reshape(BLK, 1, F)
    diff = qi3 * qj_ref[...].reshape(BLK, NB, F) - bnew3
    de = jnp.sum(diff * dinv3, axis=1)
    de = eg_ref[...] * de
    aout_o[0] = a_ref[0] + de
    eout_o[0] = ed_ref[0] + de


def _stage3(a3, qdyn2, egate2, edyn3, amsij2, aj2, qj2,
            rbf4, d3, bdyn4, blat3, p):
    f32 = jnp.float32
    atom3 = pl.BlockSpec((1, BLK, F), lambda b, i: (b, i, 0))
    atom2 = pl.BlockSpec((BLK, F), lambda b, i: (b * EB + i, 0))
    edge2 = pl.BlockSpec((CH, F), lambda b, i: (b * EB + i, 0))
    rbfs = pl.BlockSpec((1, BLK, NB, R), lambda b, i: (b, i, 0, 0))
    dns = pl.BlockSpec((1, BLK, NB), lambda b, i: (b, i, 0))
    bd4s = pl.BlockSpec((1, BLK, NB, F), lambda b, i: (b, i, 0, 0))

    def wspec(w):
        nd = len(w.shape)
        return pl.BlockSpec(w.shape, lambda b, i, _n=nd: (0,) * _n)

    wr, wb, wm = p['rbf'], p['b'], p['bm']
    weights = [wr[0], wr[1].reshape(1, F),
               wb[0], wb[1].reshape(1, F), wb[2], wb[3].reshape(1, 1),
               wm[0], wm[1].reshape(1, F), wm[2], wm[3].reshape(1, F)]
    outs = [
        jax.ShapeDtypeStruct((B, A, F), f32),
        jax.ShapeDtypeStruct((B, A, F), f32),
        jax.ShapeDtypeStruct((B, A, NB, F), f32),
        jax.ShapeDtypeStruct((B, A, NB), f32),
    ]
    return pl.pallas_call(
        _stage3_body,
        grid=(B, EB),
        in_specs=[atom3, atom2, atom2, atom3, atom2,
                  edge2, edge2, rbfs, dns,
                  bd4s, dns] + [wspec(w) for w in weights],
        out_specs=[atom3, atom3, bd4s, dns],
        out_shape=outs,
    )(a3, qdyn2, egate2, edyn3, amsij2, aj2, qj2,
      rbf4, d3, bdyn4, blat3, *weights)


def kernel(a, q_dynamics, b_dynamics, e_dynamics, q_latent, b_latent,
           rbf, D, N, NM, params):
    f32 = jnp.float32
    a2 = a.reshape(B * A, F)
    qd2 = q_dynamics.reshape(B * A, F)
    ql2 = q_latent.reshape(B * A, 1)

    amsij, qdyn_new, qlat_new, egate = _stage1(a2, qd2, ql2, params)

    idx = (N.astype(jnp.int32).reshape(B, A * NB)
           + (jnp.arange(B, dtype=jnp.int32) * A)[:, None]).reshape(B * A * NB)
    aj, qj = _sc_gather2(amsij, qdyn_new, idx)

    aout, edyn_new, bdyn_new, blat_new = _stage3(
        a,
        qdyn_new,
        egate,
        e_dynamics,
        amsij,
        aj,
        qj,
        rbf,
        D,
        b_dynamics,
        b_latent,
        params,
    )

    return (aout,
            qdyn_new.reshape(B, A, F),
            bdyn_new,
            edyn_new,
            qlat_new.reshape(B, A, 1),
            blat_new)


# speedup vs baseline: 1.1030x; 1.1030x over previous
import functools

import jax
import jax.numpy as jnp
from jax import lax
from jax.experimental import pallas as pl
from jax.experimental.pallas import tpu as pltpu
from jax.experimental.pallas import tpu_sc as plsc

B, A, NB, F, R = 4, 512, 32, 128, 20
CUTOFF = 5.0
BLK = 128
EB = A // BLK
CH = BLK * NB


def _silu(x):
    return x / (1.0 + jnp.exp(-x))


def _cutoff(D):
    x = D * (1.0 / CUTOFF)
    x2 = x * x
    x4 = x2 * x2
    x8 = x4 * x4
    x9 = x8 * x
    x10 = x9 * x
    x11 = x10 * x
    out = 1.0 - 55.0 * x9 + 99.0 * x10 - 45.0 * x11
    return out * (D < CUTOFF).astype(D.dtype)


def _mlp2(x, W1, b1, W2, b2):
    h = jnp.dot(x, W1, preferred_element_type=jnp.float32) + b1
    h = _silu(h)
    return jnp.dot(h, W2, preferred_element_type=jnp.float32) + b2


def _stage1_body(a_ref, qd_ref, ql_ref,
                 Wa1, ba1, Wa2, ba2,
                 Wq1, bq1, Wq2, bq2,
                 Wm1, bm1, Wm2, bm2,
                 We1, be1, We2, be2,
                 amsij_o, qdyn_o, qlat_o, egate_o):
    a = a_ref[...]
    amsij_o[...] = _mlp2(a, Wa1[...], ba1[...], Wa2[...], ba2[...])
    q = _mlp2(a, Wq1[...], bq1[...], Wq2[...], bq2[...])
    qm = _mlp2(a, Wm1[...], bm1[...], Wm2[...], bm2[...])
    qlat_o[...] = ql_ref[...] + q
    qdyn_o[...] = qd_ref[...] + q * qm
    egate_o[...] = _mlp2(a, We1[...], be1[...], We2[...], be2[...])


def _stage1(a2, qd2, ql2, p):
    n = a2.shape[0]
    f32 = jnp.float32
    outs = [
        jax.ShapeDtypeStruct((n, F), f32),
        jax.ShapeDtypeStruct((n, F), f32),
        jax.ShapeDtypeStruct((n, 1), f32),
        jax.ShapeDtypeStruct((n, F), f32),
    ]
    wa = p['a']
    wq = p['q']
    wm = p['qm']
    we = p['e']
    return pl.pallas_call(_stage1_body, out_shape=outs)(
        a2, qd2, ql2,
        wa[0], wa[1].reshape(1, F), wa[2], wa[3].reshape(1, F),
        wq[0], wq[1].reshape(1, F), wq[2], wq[3].reshape(1, 1),
        wm[0], wm[1].reshape(1, F), wm[2], wm[3].reshape(1, F),
        we[0], we[1].reshape(1, F), we[2], we[3].reshape(1, F),
    )


def _sc_gather2(table_a, table_b, idx):
    info = plsc.get_sparse_core_info()
    nw = info.num_cores * info.num_subcores
    m = idx.shape[0]
    per_w = m // nw
    chunk = 128
    nch = per_w // chunk
    f32 = jnp.float32

    @functools.partial(
        pl.kernel,
        out_type=[jax.ShapeDtypeStruct((m, F), f32),
                  jax.ShapeDtypeStruct((m, F), f32)],
        mesh=plsc.VectorSubcoreMesh(core_axis_name="c", subcore_axis_name="s"),
        scratch_types=[
            pltpu.VMEM((per_w,), jnp.int32),
            pltpu.VMEM((2, chunk, F), f32),
            pltpu.VMEM((2, chunk, F), f32),
            pltpu.SemaphoreType.DMA,
            pltpu.SemaphoreType.DMA,
            pltpu.SemaphoreType.DMA,
            pltpu.SemaphoreType.DMA,
        ],
    )
    def k(ta, tb, ix, out_a, out_b, idx_v, rows_a, rows_b, gsem, ssem,
          gsem2, ssem2):
        wid = lax.axis_index("s") * info.num_cores + lax.axis_index("c")
        base = wid * per_w
        pltpu.sync_copy(ix.at[pl.ds(base, per_w)], idx_v)
        gsems = (gsem, gsem2)
        ssems = (ssem, ssem2)
        scat = [None, None, None, None]
        for c in range(nch):
            s = c % 2
            if scat[2 * s] is not None:
                scat[2 * s].wait()
                scat[2 * s + 1].wait()
            isl = idx_v.at[pl.ds(c * chunk, chunk)]
            cp_a = pltpu.async_copy(ta.at[isl], rows_a.at[s], gsems[s])
            cp_b = pltpu.async_copy(tb.at[isl], rows_b.at[s], gsems[s])
            osl = pl.ds(base + c * chunk, chunk)
            cp_a.wait()
            scat[2 * s] = pltpu.async_copy(rows_a.at[s], out_a.at[osl],
                                           ssems[s])
            cp_b.wait()
            scat[2 * s + 1] = pltpu.async_copy(rows_b.at[s], out_b.at[osl],
                                               ssems[s])
        for cp in scat:
            cp.wait()

    return k(table_a, table_b, idx)


def _stage3_body(a_ref, qd_ref, eg_ref, ed_ref, ai_ref,
                 aj_ref, qj_ref, rbf_ref, d_ref,
                 bd_ref, bl_ref,
                 Wr, br, Wb1, bb1, Wb2, bb2, Wm1, bm1, Wm2, bm2,
                 aout_o, eout_o, bdyn_o, blat_o):
    d2t = d_ref[0]
    d2 = jnp.transpose(d2t, (1, 0))
    cut3 = _cutoff(d2)[:, :, None]
    dinv3 = jnp.where(d2 > 0.0, 1.0 / d2, 0.0)[:, :, None]
    rbftb = rbf_ref[0]
    rbfmT = jax.lax.dot_general(rbftb, Wr[...], (((0,), (0,)), ((), ())),
                                preferred_element_type=jnp.float32)
    rbfm3 = jnp.transpose(rbfmT, (1, 0, 2)) + br[...]
    ai3 = ai_ref[...].reshape(BLK, 1, F)
    msij3 = ai3 * aj_ref[...].reshape(BLK, NB, F) * (rbfm3 * cut3)
    msij = msij3.reshape(CH, F)
    h = _silu(jnp.dot(msij, Wb1[...], preferred_element_type=jnp.float32) + bb1[...])
    bij = jnp.dot(h, Wb2[...], preferred_element_type=jnp.float32) + bb2[...]
    h2 = _silu(jnp.dot(msij, Wm1[...], preferred_element_type=jnp.float32) + bm1[...])
    bm = jnp.dot(h2, Wm2[...], preferred_element_type=jnp.float32) + bm2[...]
    bnew3 = bd_ref[0] + (bij * bm).reshape(BLK, NB, F)
    bdyn_o[0] = bnew3
    blat_o[0] = bl_ref[0] + jnp.transpose(bij.reshape(BLK, NB), (1, 0))
    qi3 = qd_ref[...].reshape(BLK, 1, F)
    diff = qi3 * qj_ref[...].reshape(BLK, NB, F) - bnew3
    de = jnp.sum(diff * dinv3, axis=1)
    de = eg_ref[...] * de
    aout_o[0] = a_ref[0] + de
    eout_o[0] = ed_ref[0] + de


def _stage3(a3, qdyn2, egate2, edyn3, amsij2, aj2, qj2,
            rbf4, d3, bdyn4, blat3, p):
    f32 = jnp.float32
    atom3 = pl.BlockSpec((1, BLK, F), lambda b, i: (b, i, 0))
    atom2 = pl.BlockSpec((BLK, F), lambda b, i: (b * EB + i, 0))
    edge2 = pl.BlockSpec((CH, F), lambda b, i: (b * EB + i, 0))
    rbfs = pl.BlockSpec((1, R, NB, BLK), lambda b, i: (b, 0, 0, i))
    dns = pl.BlockSpec((1, NB, BLK), lambda b, i: (b, 0, i))
    bd4s = pl.BlockSpec((1, BLK, NB, F), lambda b, i: (b, i, 0, 0))

    def wspec(w):
        nd = len(w.shape)
        return pl.BlockSpec(w.shape, lambda b, i, _n=nd: (0,) * _n)

    wr, wb, wm = p['rbf'], p['b'], p['bm']
    weights = [wr[0], wr[1].reshape(1, F),
               wb[0], wb[1].reshape(1, F), wb[2], wb[3].reshape(1, 1),
               wm[0], wm[1].reshape(1, F), wm[2], wm[3].reshape(1, F)]
    outs = [
        jax.ShapeDtypeStruct((B, A, F), f32),
        jax.ShapeDtypeStruct((B, A, F), f32),
        jax.ShapeDtypeStruct((B, A, NB, F), f32),
        jax.ShapeDtypeStruct((B, NB, A), f32),
    ]
    return pl.pallas_call(
        _stage3_body,
        grid=(B, EB),
        in_specs=[atom3, atom2, atom2, atom3, atom2,
                  edge2, edge2, rbfs, dns,
                  bd4s, dns] + [wspec(w) for w in weights],
        out_specs=[atom3, atom3, bd4s, dns],
        out_shape=outs,
    )(a3, qdyn2, egate2, edyn3, amsij2, aj2, qj2,
      rbf4, d3, bdyn4, blat3, *weights)


def kernel(a, q_dynamics, b_dynamics, e_dynamics, q_latent, b_latent,
           rbf, D, N, NM, params):
    f32 = jnp.float32
    a2 = a.reshape(B * A, F)
    qd2 = q_dynamics.reshape(B * A, F)
    ql2 = q_latent.reshape(B * A, 1)

    amsij, qdyn_new, qlat_new, egate = _stage1(a2, qd2, ql2, params)

    idx = (N.astype(jnp.int32).reshape(B, A * NB)
           + (jnp.arange(B, dtype=jnp.int32) * A)[:, None]).reshape(B * A * NB)
    aj, qj = _sc_gather2(amsij, qdyn_new, idx)

    aout, edyn_new, bdyn_new, blat_new = _stage3(
        a,
        qdyn_new,
        egate,
        e_dynamics,
        amsij,
        aj,
        qj,
        rbf.transpose(0, 3, 2, 1),
        D.transpose(0, 2, 1),
        b_dynamics,
        b_latent.transpose(0, 2, 1),
        params,
    )

    return (aout,
            qdyn_new.reshape(B, A, F),
            bdyn_new,
            edyn_new,
            qlat_new.reshape(B, A, 1),
            blat_new.transpose(0, 2, 1))
